# baseline (speedup 1.0000x reference)
import functools

import jax
import jax.numpy as jnp
from jax import lax
from jax.experimental import pallas as pl
from jax.experimental.pallas import tpu as pltpu
from jax.experimental.pallas import tpu_sc as plsc

TOPK_NUM = 64
TEMP = 0.1
EPS = 1e-20

_NC = 2
_NS = 16
_L = 16


def _zt_body(x_ref, w_ref, ut_ref, zt_ref):
    x = x_ref[0]
    Wg = w_ref[...]
    ut = ut_ref[0, 0]
    mt = lax.dot_general(
        x, Wg, (((0,), (1,)), ((), ())), preferred_element_type=jnp.float32
    )
    g = -jnp.log(-jnp.log(ut + EPS) + EPS)
    zt_ref[0, 0] = mt + g


def _combine_body(x_ref, w_ref, xc_ref, out_ref):
    x = x_ref[0]
    W = w_ref[...]
    xc = xc_ref[0]
    c = x.shape[0]
    mapping = lax.dot_general(
        W, x, (((1,), (0,)), ((), ())), preferred_element_type=jnp.float32
    )
    mmax = jnp.max(mapping, axis=0, keepdims=True)
    e = jnp.exp(mapping - mmax)
    mp = e / jnp.sum(e, axis=0, keepdims=True)
    res = lax.dot_general(
        xc, mp, (((0,), (0,)), ((), ())), preferred_element_type=jnp.float32
    )
    out_ref[0] = res[:c, :]


def _make_sc_argmax_gather(n, k, hw, cp):
    groups = (n * k) // _L
    gpb = k // _L
    ch = 256

    mesh = plsc.VectorSubcoreMesh(
        core_axis_name="c",
        subcore_axis_name="s",
        num_cores=_NC,
        num_subcores=_NS,
    )

    @functools.partial(
        pl.kernel,
        mesh=mesh,
        out_type=jax.ShapeDtypeStruct((n * k, cp), jnp.float32),
        scratch_types=[
            pltpu.VMEM((ch, _L), jnp.float32),
            pltpu.VMEM((_L,), jnp.int32),
            pltpu.VMEM((_L, cp), jnp.float32),
            pltpu.SemaphoreType.DMA,
        ],
    )
    def sc_fn(zt_hbm, xt_hbm, xc_hbm, z_v, idx_v, rows_v, sem):
        wid = lax.axis_index("s") * _NC + lax.axis_index("c")

        @pl.when(wid < groups)
        def _():
            n_i = wid // gpb

            carry = (
                jnp.full((_L,), -jnp.inf, jnp.float32),
                jnp.zeros((_L,), jnp.int32),
            )
            for ci in range(hw // ch):
                pltpu.sync_copy(zt_hbm.at[wid, pl.ds(ci * ch, ch)], z_v)

                def body(p, carry, base=ci * ch):
                    vmax, vidx = carry
                    chunk = z_v[p, :]
                    upd = chunk > vmax
                    vmax = jnp.where(upd, chunk, vmax)
                    vidx = jnp.where(upd, base + p, vidx)
                    return vmax, vidx

                carry = lax.fori_loop(0, ch, body, carry)
            _, vidx = carry

            idx_v[...] = vidx + n_i * hw
            pltpu.async_copy(xt_hbm.at[idx_v], rows_v, sem).wait()
            pltpu.sync_copy(rows_v, xc_hbm.at[pl.ds(wid * _L, _L)])

    return sc_fn


def kernel(x, W, U):
    n, c, h, w = x.shape
    k = W.shape[0]
    hw = h * w
    gpb = k // _L
    cp = 128
    x2 = x.reshape(n, c, hw)
    UTg = U.reshape(n, gpb, _L, hw).transpose(0, 1, 3, 2)
    xT = jnp.pad(x2, ((0, 0), (0, cp - c), (0, 0))).transpose(0, 2, 1)
    xT = xT.reshape(n * hw, cp)

    zt = pl.pallas_call(
        _zt_body,
        grid=(n, gpb),
        in_specs=[
            pl.BlockSpec((1, c, hw), lambda i, g: (i, 0, 0)),
            pl.BlockSpec((_L, c), lambda i, g: (g, 0)),
            pl.BlockSpec((1, 1, hw, _L), lambda i, g: (i, g, 0, 0)),
        ],
        out_specs=pl.BlockSpec((1, 1, hw, _L), lambda i, g: (i, g, 0, 0)),
        out_shape=jax.ShapeDtypeStruct((n, gpb, hw, _L), jnp.float32),
    )(x2, W, UTg)

    sc_fn = _make_sc_argmax_gather(n, k, hw, cp)
    xc = sc_fn(zt.reshape(n * gpb, hw, _L), xT)

    out = pl.pallas_call(
        _combine_body,
        grid=(n,),
        in_specs=[
            pl.BlockSpec((1, c, hw), lambda i: (i, 0, 0)),
            pl.BlockSpec((k, c), lambda i: (0, 0)),
            pl.BlockSpec((1, k, cp), lambda i: (i, 0, 0)),
        ],
        out_specs=pl.BlockSpec((1, c, hw), lambda i: (i, 0, 0)),
        out_shape=jax.ShapeDtypeStruct((n, c, hw), jnp.float32),
    )(x2, W, xc.reshape(n, k, cp))
    return out.reshape(n, c, h, w)


# speedup vs baseline: 1.5579x; 1.5579x over previous
import functools

import jax
import jax.numpy as jnp
from jax import lax
from jax.experimental import pallas as pl
from jax.experimental.pallas import tpu as pltpu
from jax.experimental.pallas import tpu_sc as plsc

TOPK_NUM = 64
TEMP = 0.1
EPS = 1e-20

_NC = 2
_NS = 16
_L = 16


def _stage_a_body(x_ref, w_ref, u_ref, zt_ref, xt_ref):
    x = x_ref[0]
    W = w_ref[...]
    u = u_ref[0]
    c, hw = x.shape
    cp = xt_ref.shape[2]
    mapping = lax.dot_general(
        W, x, (((1,), (0,)), ((), ())), preferred_element_type=jnp.float32
    )
    g = -jnp.log(-jnp.log(u + EPS) + EPS)
    zt_ref[0] = jnp.transpose(mapping + g)
    xt = jnp.transpose(x)
    xt_ref[0] = jnp.concatenate(
        [xt, jnp.zeros((hw, cp - c), jnp.float32)], axis=1
    )


def _stage_c_body(x_ref, w_ref, xc_ref, out_ref):
    x = x_ref[0]
    W = w_ref[...]
    xc = xc_ref[0]
    c = x.shape[0]
    mapping = lax.dot_general(
        W, x, (((1,), (0,)), ((), ())), preferred_element_type=jnp.float32
    )
    mmax = jnp.max(mapping, axis=0, keepdims=True)
    e = jnp.exp(mapping - mmax)
    mp = e / jnp.sum(e, axis=0, keepdims=True)
    res = lax.dot_general(
        xc, mp, (((0,), (0,)), ((), ())), preferred_element_type=jnp.float32
    )
    out_ref[0] = res[:c, :]


def _make_sc_argmax_gather(n, k, hw, cp):
    gpk = k // _L
    ch = 256
    nch = hw // ch

    mesh = plsc.VectorSubcoreMesh(
        core_axis_name="c",
        subcore_axis_name="s",
        num_cores=_NC,
        num_subcores=_NS,
    )

    @functools.partial(
        pl.kernel,
        mesh=mesh,
        out_type=jax.ShapeDtypeStruct((n * k, cp), jnp.float32),
        scratch_types=[
            pltpu.VMEM((2, ch, k), jnp.float32),
            pltpu.VMEM((k,), jnp.int32),
            pltpu.VMEM((k, cp), jnp.float32),
            pltpu.SemaphoreType.DMA((2,)),
            pltpu.SemaphoreType.DMA,
        ],
    )
    def sc_fn(zt_hbm, xt_hbm, xc_hbm, z_v, idx_v, rows_v, copy_sems, gsem):
        wid = lax.axis_index("s") * _NC + lax.axis_index("c")

        @pl.when(wid < n)
        def _():
            def start_copy(ci, slot):
                pltpu.async_copy(
                    zt_hbm.at[wid, pl.ds(ci * ch, ch)],
                    z_v.at[slot],
                    copy_sems.at[slot],
                )

            start_copy(0, 0)

            carries = [
                (
                    jnp.full((_L,), -jnp.inf, jnp.float32),
                    jnp.zeros((_L,), jnp.int32),
                )
                for _ in range(gpk)
            ]

            for ci in range(nch):
                slot = ci % 2
                if ci + 1 < nch:
                    start_copy(ci + 1, 1 - slot)
                pltpu.make_async_copy(
                    zt_hbm.at[wid, pl.ds(ci * ch, ch)],
                    z_v.at[slot],
                    copy_sems.at[slot],
                ).wait()

                def body(p, carry, base=ci * ch, slot=slot):
                    new = []
                    for g in range(gpk):
                        vmax, vidx = carry[2 * g], carry[2 * g + 1]
                        chunk = z_v[slot, p, pl.ds(g * _L, _L)]
                        upd = chunk > vmax
                        vmax = jnp.where(upd, chunk, vmax)
                        vidx = jnp.where(upd, base + p, vidx)
                        new.extend((vmax, vidx))
                    return tuple(new)

                flat = lax.fori_loop(
                    0, ch, body, tuple(v for cr in carries for v in cr)
                )
                carries = [
                    (flat[2 * g], flat[2 * g + 1]) for g in range(gpk)
                ]

            for g in range(gpk):
                idx_v[pl.ds(g * _L, _L)] = carries[g][1] + wid * hw

            pltpu.async_copy(xt_hbm.at[idx_v], rows_v, gsem).wait()
            pltpu.sync_copy(rows_v, xc_hbm.at[pl.ds(wid * k, k)])

    return sc_fn


def kernel(x, W, U):
    n, c, h, w = x.shape
    k = W.shape[0]
    hw = h * w
    cp = 128
    x2 = x.reshape(n, c, hw)
    U2 = U.reshape(n, k, hw)

    zt, xt = pl.pallas_call(
        _stage_a_body,
        grid=(n,),
        in_specs=[
            pl.BlockSpec((1, c, hw), lambda i: (i, 0, 0)),
            pl.BlockSpec((k, c), lambda i: (0, 0)),
            pl.BlockSpec((1, k, hw), lambda i: (i, 0, 0)),
        ],
        out_specs=[
            pl.BlockSpec((1, hw, k), lambda i: (i, 0, 0)),
            pl.BlockSpec((1, hw, cp), lambda i: (i, 0, 0)),
        ],
        out_shape=[
            jax.ShapeDtypeStruct((n, hw, k), jnp.float32),
            jax.ShapeDtypeStruct((n, hw, cp), jnp.float32),
        ],
    )(x2, W, U2)

    sc_fn = _make_sc_argmax_gather(n, k, hw, cp)
    xc = sc_fn(zt, xt.reshape(n * hw, cp))

    out = pl.pallas_call(
        _stage_c_body,
        grid=(n,),
        in_specs=[
            pl.BlockSpec((1, c, hw), lambda i: (i, 0, 0)),
            pl.BlockSpec((k, c), lambda i: (0, 0)),
            pl.BlockSpec((1, k, cp), lambda i: (i, 0, 0)),
        ],
        out_specs=pl.BlockSpec((1, c, hw), lambda i: (i, 0, 0)),
        out_shape=jax.ShapeDtypeStruct((n, c, hw), jnp.float32),
    )(x2, W, xc.reshape(n, k, cp))
    return out.reshape(n, c, h, w)


# speedup vs baseline: 1.6704x; 1.0722x over previous
import functools

import jax
import jax.numpy as jnp
from jax import lax
from jax.experimental import pallas as pl
from jax.experimental.pallas import tpu as pltpu
from jax.experimental.pallas import tpu_sc as plsc

TOPK_NUM = 64
TEMP = 0.1
EPS = 1e-20

_NC = 2
_NS = 16
_L = 16


def _stage_a_body(x_ref, w_ref, u_ref, zt_ref, xt_ref):
    x = x_ref[0]
    W = w_ref[...]
    u = u_ref[0]
    c, hw = x.shape
    cp = xt_ref.shape[2]
    mapping = lax.dot_general(
        W, x, (((1,), (0,)), ((), ())), preferred_element_type=jnp.float32
    )
    g = -jnp.log(-jnp.log(u + EPS) + EPS)
    zt_ref[0] = jnp.transpose(mapping + g)
    xt = jnp.transpose(x)
    xt_ref[0] = jnp.concatenate(
        [xt, jnp.zeros((hw, cp - c), jnp.float32)], axis=1
    )


def _stage_c_body(x_ref, w_ref, candv_ref, xcand_ref, out_ref):
    x = x_ref[0]
    W = w_ref[...]
    cv = candv_ref[0]
    qn = cv.shape[0]
    c = x.shape[0]
    cvT = jnp.transpose(cv)
    bestv = jnp.max(cvT, axis=1, keepdims=True)
    eq = cvT == bestv
    taken = jnp.zeros_like(bestv, dtype=jnp.bool_)
    xc = jnp.zeros_like(xcand_ref[0, 0])
    for q in range(qn):
        m = jnp.logical_and(eq[:, q:q + 1], jnp.logical_not(taken))
        taken = jnp.logical_or(taken, m)
        xc = xc + jnp.where(m, xcand_ref[0, q], 0.0)
    mapping = lax.dot_general(
        W, x, (((1,), (0,)), ((), ())), preferred_element_type=jnp.float32
    )
    mmax = jnp.max(mapping, axis=0, keepdims=True)
    e = jnp.exp(mapping - mmax)
    mp = e / jnp.sum(e, axis=0, keepdims=True)
    res = lax.dot_general(
        xc, mp, (((0,), (0,)), ((), ())), preferred_element_type=jnp.float32
    )
    out_ref[0] = res[:c, :]


def _make_sc_argmax_gather(n, k, hw, cp):
    gpk = k // _L
    bpc = n // _NC
    qn = _NS // bpc
    ch = hw // qn

    mesh = plsc.VectorSubcoreMesh(
        core_axis_name="c",
        subcore_axis_name="s",
        num_cores=_NC,
        num_subcores=_NS,
    )

    @functools.partial(
        pl.kernel,
        mesh=mesh,
        out_type=[
            jax.ShapeDtypeStruct((n * qn, k), jnp.float32),
            jax.ShapeDtypeStruct((n * qn * k, cp), jnp.float32),
        ],
        scratch_types=[
            pltpu.VMEM((ch, k), jnp.float32),
            pltpu.VMEM((k,), jnp.float32),
            pltpu.VMEM((k,), jnp.int32),
            pltpu.VMEM((k, cp), jnp.float32),
            pltpu.SemaphoreType.DMA,
            pltpu.SemaphoreType.DMA,
        ],
    )
    def sc_fn(zt_hbm, xt_hbm, candv_hbm, xcand_hbm, z_v, candv_v, idx_v,
              rows_v, zsem, gsem):
        ci = lax.axis_index("c")
        s = lax.axis_index("s")
        b_local = s // qn
        batch = ci * bpc + b_local
        q = s % qn
        shard = batch * qn + q

        pltpu.async_copy(
            zt_hbm.at[batch, pl.ds(q * ch, ch)], z_v, zsem
        ).wait()

        def body(p, carry):
            new = []
            for g in range(gpk):
                vmax, vidx = carry[2 * g], carry[2 * g + 1]
                chunk = z_v[p, pl.ds(g * _L, _L)]
                upd = chunk > vmax
                vmax = jnp.where(upd, chunk, vmax)
                vidx = jnp.where(upd, p, vidx)
                new.extend((vmax, vidx))
            return tuple(new)

        init = []
        for _ in range(gpk):
            init.append(jnp.full((_L,), -jnp.inf, jnp.float32))
            init.append(jnp.zeros((_L,), jnp.int32))
        flat = lax.fori_loop(0, ch, body, tuple(init))

        for g in range(gpk):
            candv_v[pl.ds(g * _L, _L)] = flat[2 * g]
            idx_v[pl.ds(g * _L, _L)] = (
                flat[2 * g + 1] + q * ch + batch * hw
            )

        pltpu.async_copy(xt_hbm.at[idx_v], rows_v, gsem).wait()
        pltpu.sync_copy(rows_v, xcand_hbm.at[pl.ds(shard * k, k)])
        pltpu.sync_copy(candv_v, candv_hbm.at[shard])

    return sc_fn


def kernel(x, W, U):
    n, c, h, w = x.shape
    k = W.shape[0]
    hw = h * w
    cp = 128
    x2 = x.reshape(n, c, hw)
    U2 = U.reshape(n, k, hw)

    zt, xt = pl.pallas_call(
        _stage_a_body,
        grid=(n,),
        in_specs=[
            pl.BlockSpec((1, c, hw), lambda i: (i, 0, 0)),
            pl.BlockSpec((k, c), lambda i: (0, 0)),
            pl.BlockSpec((1, k, hw), lambda i: (i, 0, 0)),
        ],
        out_specs=[
            pl.BlockSpec((1, hw, k), lambda i: (i, 0, 0)),
            pl.BlockSpec((1, hw, cp), lambda i: (i, 0, 0)),
        ],
        out_shape=[
            jax.ShapeDtypeStruct((n, hw, k), jnp.float32),
            jax.ShapeDtypeStruct((n, hw, cp), jnp.float32),
        ],
    )(x2, W, U2)

    qn = (_NC * _NS) // n
    sc_fn = _make_sc_argmax_gather(n, k, hw, cp)
    candv, xcand = sc_fn(zt, xt.reshape(n * hw, cp))

    out = pl.pallas_call(
        _stage_c_body,
        grid=(n,),
        in_specs=[
            pl.BlockSpec((1, c, hw), lambda i: (i, 0, 0)),
            pl.BlockSpec((k, c), lambda i: (0, 0)),
            pl.BlockSpec((1, qn, k), lambda i: (i, 0, 0)),
            pl.BlockSpec((1, qn, k, cp), lambda i: (i, 0, 0, 0)),
        ],
        out_specs=pl.BlockSpec((1, c, hw), lambda i: (i, 0, 0)),
        out_shape=jax.ShapeDtypeStruct((n, c, hw), jnp.float32),
    )(x2, W, candv.reshape(n, qn, k), xcand.reshape(n, qn, k, cp))
    return out.reshape(n, c, h, w)


# speedup vs baseline: 1.7515x; 1.0486x over previous
import functools

import jax
import jax.numpy as jnp
from jax import lax
from jax.experimental import pallas as pl
from jax.experimental.pallas import tpu as pltpu
from jax.experimental.pallas import tpu_sc as plsc

TOPK_NUM = 64
TEMP = 0.1
EPS = 1e-20

_NC = 2
_NS = 16
_L = 16


def _stage_a_body(x_ref, w_ref, u_ref, zt_ref, xt_ref):
    W = w_ref[...]
    n, c, hw = x_ref.shape
    cp = xt_ref.shape[2]
    for b in range(n):
        x = x_ref[b]
        u = u_ref[b]
        mapping = lax.dot_general(
            W, x, (((1,), (0,)), ((), ())),
            preferred_element_type=jnp.float32,
        )
        g = -jnp.log(-jnp.log(u + EPS) + EPS)
        zt_ref[b] = jnp.transpose(mapping + g)
        xt = jnp.transpose(x)
        xt_ref[b] = jnp.concatenate(
            [xt, jnp.zeros((hw, cp - c), jnp.float32)], axis=1
        )


def _stage_c_body(x_ref, w_ref, candv_ref, xcand_ref, out_ref):
    W = w_ref[...]
    n, c, hw = x_ref.shape
    qn = candv_ref.shape[1]
    for b in range(n):
        x = x_ref[b]
        cv = candv_ref[b]
        cvT = jnp.transpose(cv)
        bestv = jnp.max(cvT, axis=1, keepdims=True)
        eq = cvT == bestv
        taken = jnp.zeros_like(bestv, dtype=jnp.bool_)
        xc = jnp.zeros_like(xcand_ref[0, 0])
        for q in range(qn):
            m = jnp.logical_and(eq[:, q:q + 1], jnp.logical_not(taken))
            taken = jnp.logical_or(taken, m)
            xc = xc + jnp.where(m, xcand_ref[b, q], 0.0)
        mapping = lax.dot_general(
            W, x, (((1,), (0,)), ((), ())),
            preferred_element_type=jnp.float32,
        )
        mmax = jnp.max(mapping, axis=0, keepdims=True)
        e = jnp.exp(mapping - mmax)
        mp = e / jnp.sum(e, axis=0, keepdims=True)
        res = lax.dot_general(
            xc, mp, (((0,), (0,)), ((), ())),
            preferred_element_type=jnp.float32,
        )
        out_ref[b] = res[:c, :]


def _make_sc_argmax_gather(n, k, hw, cp):
    gpk = k // _L
    bpc = n // _NC
    qn = _NS // bpc
    ch = hw // qn

    mesh = plsc.VectorSubcoreMesh(
        core_axis_name="c",
        subcore_axis_name="s",
        num_cores=_NC,
        num_subcores=_NS,
    )

    @functools.partial(
        pl.kernel,
        mesh=mesh,
        out_type=[
            jax.ShapeDtypeStruct((n * qn, k), jnp.float32),
            jax.ShapeDtypeStruct((n * qn * k, cp), jnp.float32),
        ],
        scratch_types=[
            pltpu.VMEM((ch, k), jnp.float32),
            pltpu.VMEM((k,), jnp.float32),
            pltpu.VMEM((k,), jnp.int32),
            pltpu.VMEM((k, cp), jnp.float32),
            pltpu.SemaphoreType.DMA,
            pltpu.SemaphoreType.DMA,
        ],
    )
    def sc_fn(zt_hbm, xt_hbm, candv_hbm, xcand_hbm, z_v, candv_v, idx_v,
              rows_v, zsem, gsem):
        ci = lax.axis_index("c")
        s = lax.axis_index("s")
        b_local = s // qn
        batch = ci * bpc + b_local
        q = s % qn
        shard = batch * qn + q

        pltpu.async_copy(
            zt_hbm.at[batch, pl.ds(q * ch, ch)], z_v, zsem
        ).wait()

        def body(p, carry):
            new = []
            for g in range(gpk):
                vmax, vidx = carry[2 * g], carry[2 * g + 1]
                chunk = z_v[p, pl.ds(g * _L, _L)]
                upd = chunk > vmax
                vmax = jnp.where(upd, chunk, vmax)
                vidx = jnp.where(upd, p, vidx)
                new.extend((vmax, vidx))
            return tuple(new)

        init = []
        for _ in range(gpk):
            init.append(jnp.full((_L,), -jnp.inf, jnp.float32))
            init.append(jnp.zeros((_L,), jnp.int32))
        flat = lax.fori_loop(0, ch, body, tuple(init))

        for g in range(gpk):
            candv_v[pl.ds(g * _L, _L)] = flat[2 * g]
            idx_v[pl.ds(g * _L, _L)] = (
                flat[2 * g + 1] + q * ch + batch * hw
            )

        pltpu.async_copy(xt_hbm.at[idx_v], rows_v, gsem).wait()
        pltpu.sync_copy(rows_v, xcand_hbm.at[pl.ds(shard * k, k)])
        pltpu.sync_copy(candv_v, candv_hbm.at[shard])

    return sc_fn


def kernel(x, W, U):
    n, c, h, w = x.shape
    k = W.shape[0]
    hw = h * w
    cp = 128
    x2 = x.reshape(n, c, hw)
    U2 = U.reshape(n, k, hw)

    zt, xt = pl.pallas_call(
        _stage_a_body,
        out_shape=[
            jax.ShapeDtypeStruct((n, hw, k), jnp.float32),
            jax.ShapeDtypeStruct((n, hw, cp), jnp.float32),
        ],
    )(x2, W, U2)

    qn = (_NC * _NS) // n
    sc_fn = _make_sc_argmax_gather(n, k, hw, cp)
    candv, xcand = sc_fn(zt, xt.reshape(n * hw, cp))

    out = pl.pallas_call(
        _stage_c_body,
        out_shape=jax.ShapeDtypeStruct((n, c, hw), jnp.float32),
    )(x2, W, candv.reshape(n, qn, k), xcand.reshape(n, qn, k, cp))
    return out.reshape(n, c, h, w)


# speedup vs baseline: 1.8482x; 1.0552x over previous
import functools

import jax
import jax.numpy as jnp
from jax import lax
from jax.experimental import pallas as pl
from jax.experimental.pallas import tpu as pltpu
from jax.experimental.pallas import tpu_sc as plsc

TOPK_NUM = 64
TEMP = 0.1
EPS = 1e-20

_NC = 2
_NS = 16
_L = 16


def _stage_a_body(x_ref, w_ref, ut_ref, zt_ref, xt_ref):
    W = w_ref[...]
    n, c, hw = x_ref.shape
    cp = xt_ref.shape[2]
    for b in range(n):
        x = x_ref[b]
        ut = ut_ref[b]
        xt = jnp.transpose(x)
        mt = lax.dot_general(
            xt, W, (((1,), (1,)), ((), ())),
            preferred_element_type=jnp.float32,
        )
        g = -jnp.log(-jnp.log(ut + EPS) + EPS)
        zt_ref[b] = mt + g
        xt_ref[b] = jnp.concatenate(
            [xt, jnp.zeros((hw, cp - c), jnp.float32)], axis=1
        )


def _stage_c_body(x_ref, w_ref, candv_ref, xcand_ref, out_ref):
    W = w_ref[...]
    n, c, hw = x_ref.shape
    qn = candv_ref.shape[1]
    for b in range(n):
        x = x_ref[b]
        cv = candv_ref[b]
        cvT = jnp.transpose(cv)
        bestv = jnp.max(cvT, axis=1, keepdims=True)
        eq = cvT == bestv
        taken = jnp.zeros_like(bestv, dtype=jnp.bool_)
        xc = jnp.zeros_like(xcand_ref[0, 0])
        for q in range(qn):
            m = jnp.logical_and(eq[:, q:q + 1], jnp.logical_not(taken))
            taken = jnp.logical_or(taken, m)
            xc = xc + jnp.where(m, xcand_ref[b, q], 0.0)
        mapping = lax.dot_general(
            W, x, (((1,), (0,)), ((), ())),
            preferred_element_type=jnp.float32,
        )
        mmax = jnp.max(mapping, axis=0, keepdims=True)
        e = jnp.exp(mapping - mmax)
        mp = e / jnp.sum(e, axis=0, keepdims=True)
        res = lax.dot_general(
            xc, mp, (((0,), (0,)), ((), ())),
            preferred_element_type=jnp.float32,
        )
        out_ref[b] = res[:c, :]


def _make_sc_argmax_gather(n, k, hw, cp):
    gpk = k // _L
    bpc = n // _NC
    qn = _NS // bpc
    ch = hw // qn

    mesh = plsc.VectorSubcoreMesh(
        core_axis_name="c",
        subcore_axis_name="s",
        num_cores=_NC,
        num_subcores=_NS,
    )

    @functools.partial(
        pl.kernel,
        mesh=mesh,
        out_type=[
            jax.ShapeDtypeStruct((n * qn, k), jnp.float32),
            jax.ShapeDtypeStruct((n * qn * k, cp), jnp.float32),
        ],
        scratch_types=[
            pltpu.VMEM((ch, k), jnp.float32),
            pltpu.VMEM((k,), jnp.float32),
            pltpu.VMEM((k,), jnp.int32),
            pltpu.VMEM((k, cp), jnp.float32),
            pltpu.SemaphoreType.DMA,
            pltpu.SemaphoreType.DMA,
        ],
    )
    def sc_fn(zt_hbm, xt_hbm, candv_hbm, xcand_hbm, z_v, candv_v, idx_v,
              rows_v, zsem, gsem):
        ci = lax.axis_index("c")
        s = lax.axis_index("s")
        b_local = s // qn
        batch = ci * bpc + b_local
        q = s % qn
        shard = batch * qn + q

        pltpu.async_copy(
            zt_hbm.at[batch, pl.ds(q * ch, ch)], z_v, zsem
        ).wait()

        def body(p, carry):
            new = []
            for g in range(gpk):
                vmax, vidx = carry[2 * g], carry[2 * g + 1]
                chunk = z_v[p, pl.ds(g * _L, _L)]
                upd = chunk > vmax
                vmax = jnp.where(upd, chunk, vmax)
                vidx = jnp.where(upd, p, vidx)
                new.extend((vmax, vidx))
            return tuple(new)

        init = []
        for _ in range(gpk):
            init.append(jnp.full((_L,), -jnp.inf, jnp.float32))
            init.append(jnp.zeros((_L,), jnp.int32))
        flat = lax.fori_loop(0, ch, body, tuple(init))

        for g in range(gpk):
            candv_v[pl.ds(g * _L, _L)] = flat[2 * g]
            idx_v[pl.ds(g * _L, _L)] = (
                flat[2 * g + 1] + q * ch + batch * hw
            )

        pltpu.async_copy(xt_hbm.at[idx_v], rows_v, gsem).wait()
        pltpu.sync_copy(rows_v, xcand_hbm.at[pl.ds(shard * k, k)])
        pltpu.sync_copy(candv_v, candv_hbm.at[shard])

    return sc_fn


def kernel(x, W, U):
    n, c, h, w = x.shape
    k = W.shape[0]
    hw = h * w
    cp = 128
    x2 = x.reshape(n, c, hw)
    U2t = U.reshape(n, k, hw).transpose(0, 2, 1)

    zt, xt = pl.pallas_call(
        _stage_a_body,
        out_shape=[
            jax.ShapeDtypeStruct((n, hw, k), jnp.float32),
            jax.ShapeDtypeStruct((n, hw, cp), jnp.float32),
        ],
    )(x2, W, U2t)

    qn = (_NC * _NS) // n
    sc_fn = _make_sc_argmax_gather(n, k, hw, cp)
    candv, xcand = sc_fn(zt, xt.reshape(n * hw, cp))

    out = pl.pallas_call(
        _stage_c_body,
        out_shape=jax.ShapeDtypeStruct((n, c, hw), jnp.float32),
    )(x2, W, candv.reshape(n, qn, k), xcand.reshape(n, qn, k, cp))
    return out.reshape(n, c, h, w)
